# baseline (speedup 1.0000x reference)
import functools

import jax
import jax.numpy as jnp
from jax import lax
from jax.experimental import pallas as pl
from jax.experimental.pallas import tpu as pltpu
from jax.experimental.pallas import tpu_sc as plsc

NW = 32
CHUNK = 40


def _make_gather(ntok: int, vocab: int, dim: int, dim_pad: int):
    bpw = ntok // NW
    nch = bpw // CHUNK
    mesh = plsc.VectorSubcoreMesh(core_axis_name="c", subcore_axis_name="s")

    @functools.partial(
        pl.kernel,
        mesh=mesh,
        out_type=jax.ShapeDtypeStruct((ntok, dim), jnp.float32),
        scratch_types=[
            pltpu.VMEM((bpw,), jnp.int32),
            pltpu.VMEM((CHUNK, dim), jnp.float32),
            pltpu.SemaphoreType.DMA,
        ],
        compiler_params=pltpu.CompilerParams(use_tc_tiling_on_sc=False),
    )
    def gather_kernel(idx_hbm, table_hbm, out_hbm, idx_v, buf, sem):
        wid = lax.axis_index("s") * 2 + lax.axis_index("c")
        base = wid * bpw
        pltpu.sync_copy(idx_hbm.at[pl.ds(base, bpw)], idx_v)

        def body(i, carry):
            start = i * CHUNK
            pltpu.async_copy(
                table_hbm.at[idx_v.at[pl.ds(start, CHUNK)]], buf, sem
            ).wait()
            pltpu.sync_copy(buf, out_hbm.at[pl.ds(base + start, CHUNK)])
            return carry

        lax.fori_loop(0, nch, body, 0)

    return gather_kernel


def kernel(input, table):
    b, t = input.shape
    vocab, dim = table.shape
    idx = input.reshape(-1).astype(jnp.int32)
    out = _make_gather(b * t, vocab, dim, dim)(idx, table)
    return out.reshape(b, t, dim)


# speedup vs baseline: 1.0244x; 1.0244x over previous
import functools

import jax
import jax.numpy as jnp
from jax import lax
from jax.experimental import pallas as pl
from jax.experimental.pallas import tpu as pltpu
from jax.experimental.pallas import tpu_sc as plsc

NW = 32
CHUNK = 40


def _make_gather(ntok: int, vocab: int, dim: int, dim_pad: int):
    bpw = ntok // NW
    nch = bpw // CHUNK
    mesh = plsc.VectorSubcoreMesh(core_axis_name="c", subcore_axis_name="s")

    npairs = nch // 2

    @functools.partial(
        pl.kernel,
        mesh=mesh,
        out_type=jax.ShapeDtypeStruct((ntok, dim), jnp.float32),
        scratch_types=[
            pltpu.VMEM((bpw,), jnp.int32),
            pltpu.VMEM((CHUNK, dim), jnp.float32),
            pltpu.VMEM((CHUNK, dim), jnp.float32),
            pltpu.SemaphoreType.DMA,
            pltpu.SemaphoreType.DMA,
            pltpu.SemaphoreType.DMA,
            pltpu.SemaphoreType.DMA,
        ],
        compiler_params=pltpu.CompilerParams(use_tc_tiling_on_sc=False),
    )
    def gather_kernel(idx_hbm, table_hbm, out_hbm, idx_v, buf0, buf1,
                      gsem0, gsem1, wsem0, wsem1):
        wid = lax.axis_index("s") * 2 + lax.axis_index("c")
        base = wid * bpw
        pltpu.sync_copy(idx_hbm.at[pl.ds(base, bpw)], idx_v)

        def gather(c, buf, gsem):
            pltpu.async_copy(table_hbm.at[idx_v.at[pl.ds(c * CHUNK, CHUNK)]],
                             buf, gsem)

        def wait_gather(buf, gsem):
            pltpu.make_async_copy(table_hbm.at[pl.ds(0, CHUNK)], buf, gsem).wait()

        def put(c, buf, wsem):
            pltpu.async_copy(buf, out_hbm.at[pl.ds(base + c * CHUNK, CHUNK)],
                             wsem)

        def wait_put(buf, wsem):
            pltpu.make_async_copy(buf, out_hbm.at[pl.ds(base, CHUNK)], wsem).wait()

        gather(0, buf0, gsem0)
        gather(1, buf1, gsem1)

        def body(j, carry):
            c0 = j * 2
            wait_gather(buf0, gsem0)
            put(c0, buf0, wsem0)
            wait_gather(buf1, gsem1)
            put(c0 + 1, buf1, wsem1)

            @pl.when(j < npairs - 1)
            def _():
                wait_put(buf0, wsem0)
                gather(c0 + 2, buf0, gsem0)
                wait_put(buf1, wsem1)
                gather(c0 + 3, buf1, gsem1)

            return carry

        lax.fori_loop(0, npairs, body, 0)
        wait_put(buf0, wsem0)
        wait_put(buf1, wsem1)

    return gather_kernel


def kernel(input, table):
    b, t = input.shape
    vocab, dim = table.shape
    idx = input.reshape(-1).astype(jnp.int32)
    out = _make_gather(b * t, vocab, dim, dim)(idx, table)
    return out.reshape(b, t, dim)
